# baseline (speedup 1.0000x reference)
import functools

import jax
import jax.numpy as jnp
from jax import lax
from jax.experimental import pallas as pl
from jax.experimental.pallas import tpu as pltpu
from jax.experimental.pallas import tpu_sc as plsc

B = 16384
D = 128
H1, H2, H3 = 1024, 512, 256

NC, NS = 2, 16
NW = NC * NS
CHUNK = 128

BT = 2048
CHUNKS = (4096, 12288)


def _gather_sc(x_u, x_i, u_emb, i_emb, nb):
    bpw = nb // NW
    nch = -(-bpw // CHUNK)
    assert bpw % nch == 0
    chunk = bpw // nch
    mesh = plsc.VectorSubcoreMesh(core_axis_name="c", subcore_axis_name="s")

    @functools.partial(
        pl.kernel,
        mesh=mesh,
        out_type=(
            jax.ShapeDtypeStruct((nb, D), jnp.float32),
            jax.ShapeDtypeStruct((nb, D), jnp.float32),
        ),
        scratch_types=[
            pltpu.VMEM((bpw,), jnp.int32),
            pltpu.VMEM((bpw, D), jnp.float32),
            pltpu.SemaphoreType.DMA,
        ],
        compiler_params=pltpu.CompilerParams(use_tc_tiling_on_sc=True),
    )
    def gk(xu_hbm, xi_hbm, ue_hbm, ie_hbm, hu_hbm, hi_hbm, idx_v, rows_v, sem):
        wid = lax.axis_index("s") * NC + lax.axis_index("c")
        base = wid * bpw
        for idx_hbm, tbl_hbm, out_hbm in (
            (xu_hbm, ue_hbm, hu_hbm),
            (xi_hbm, ie_hbm, hi_hbm),
        ):
            pltpu.sync_copy(idx_hbm.at[pl.ds(base, bpw)], idx_v)
            copies = []
            for j in range(nch):
                copies.append(pltpu.async_copy(
                    tbl_hbm.at[idx_v.at[pl.ds(j * chunk, chunk)]],
                    rows_v.at[pl.ds(j * chunk, chunk)],
                    sem,
                ))
            for c in copies:
                c.wait()
            pltpu.sync_copy(rows_v, out_hbm.at[pl.ds(base, bpw)])

    return gk(x_u, x_i, u_emb, i_emb)


def _mlp_body(hu, hi, w1, b1, w2, b2, w3, b3, wo, bo, out):
    h = jnp.concatenate([hu[...], hi[...]], axis=1)
    h1 = jnp.maximum(
        jnp.dot(h, w1[...], preferred_element_type=jnp.float32) + b1[...], 0.0)
    h2 = jnp.maximum(
        jnp.dot(h1, w2[...], preferred_element_type=jnp.float32) + b2[...], 0.0)
    h3 = jnp.maximum(
        jnp.dot(h2, w3[...], preferred_element_type=jnp.float32) + b3[...], 0.0)
    out[...] = jnp.dot(h3, wo[...], preferred_element_type=jnp.float32) + bo[...]


def _mlp_tc(hu, hi, W1, b1, W2, b2, W3, b3, Wo, bo):
    nb = hu.shape[0]
    full = lambda *shape: pl.BlockSpec(shape, lambda i: (0,) * len(shape))
    return pl.pallas_call(
        _mlp_body,
        grid=(nb // BT,),
        in_specs=[
            pl.BlockSpec((BT, D), lambda i: (i, 0)),
            pl.BlockSpec((BT, D), lambda i: (i, 0)),
            full(2 * D, H1),
            full(H1),
            full(H1, H2),
            full(H2),
            full(H2, H3),
            full(H3),
            full(H3, 1),
            full(1),
        ],
        out_specs=pl.BlockSpec((BT, 1), lambda i: (i, 0)),
        out_shape=jax.ShapeDtypeStruct((nb, 1), jnp.float32),
    )(hu, hi, W1, b1, W2, b2, W3, b3, Wo, bo)


def kernel(x_u, x_i, u_emb, i_emb, W1, b1, W2, b2, W3, b3, Wo, bo):
    x_u = x_u.astype(jnp.int32)
    x_i = x_i.astype(jnp.int32)
    outs = []
    off = 0
    for nb in CHUNKS:
        sl = slice(off, off + nb)
        hu, hi = _gather_sc(x_u[sl], x_i[sl], u_emb, i_emb, nb)
        outs.append(_mlp_tc(hu, hi, W1, b1, W2, b2, W3, b3, Wo, bo))
        off += nb
    return jnp.concatenate(outs, axis=0) if len(CHUNKS) > 1 else outs[0]


# speedup vs baseline: 1.0496x; 1.0496x over previous
import functools

import jax
import jax.numpy as jnp
from jax import lax
from jax.experimental import pallas as pl
from jax.experimental.pallas import tpu as pltpu
from jax.experimental.pallas import tpu_sc as plsc

B = 16384
D = 128
H1, H2, H3 = 1024, 512, 256

NC, NS = 2, 16
NW = NC * NS
CHUNK = 128

BT = 2048
CHUNKS = (8192, 8192)


def _gather_sc(x_u, x_i, u_emb, i_emb, nb):
    bpw = nb // NW
    nch = -(-bpw // CHUNK)
    assert bpw % nch == 0
    chunk = bpw // nch
    mesh = plsc.VectorSubcoreMesh(core_axis_name="c", subcore_axis_name="s")

    @functools.partial(
        pl.kernel,
        mesh=mesh,
        out_type=(
            jax.ShapeDtypeStruct((nb, D), jnp.float32),
            jax.ShapeDtypeStruct((nb, D), jnp.float32),
        ),
        scratch_types=[
            pltpu.VMEM((bpw,), jnp.int32),
            pltpu.VMEM((bpw, D), jnp.float32),
            pltpu.SemaphoreType.DMA,
        ],
        compiler_params=pltpu.CompilerParams(use_tc_tiling_on_sc=True),
    )
    def gk(xu_hbm, xi_hbm, ue_hbm, ie_hbm, hu_hbm, hi_hbm, idx_v, rows_v, sem):
        wid = lax.axis_index("s") * NC + lax.axis_index("c")
        base = wid * bpw
        for idx_hbm, tbl_hbm, out_hbm in (
            (xu_hbm, ue_hbm, hu_hbm),
            (xi_hbm, ie_hbm, hi_hbm),
        ):
            pltpu.sync_copy(idx_hbm.at[pl.ds(base, bpw)], idx_v)
            copies = []
            for j in range(nch):
                copies.append(pltpu.async_copy(
                    tbl_hbm.at[idx_v.at[pl.ds(j * chunk, chunk)]],
                    rows_v.at[pl.ds(j * chunk, chunk)],
                    sem,
                ))
            for c in copies:
                c.wait()
            pltpu.sync_copy(rows_v, out_hbm.at[pl.ds(base, bpw)])

    return gk(x_u, x_i, u_emb, i_emb)


def _mlp_body(hu, hi, w1, b1, w2, b2, w3, b3, wo, bo, out):
    h = jnp.concatenate([hu[...], hi[...]], axis=1)
    h1 = jnp.maximum(
        jnp.dot(h, w1[...], preferred_element_type=jnp.float32) + b1[...], 0.0)
    h2 = jnp.maximum(
        jnp.dot(h1, w2[...], preferred_element_type=jnp.float32) + b2[...], 0.0)
    h3 = jnp.maximum(
        jnp.dot(h2, w3[...], preferred_element_type=jnp.float32) + b3[...], 0.0)
    out[...] = jnp.dot(h3, wo[...], preferred_element_type=jnp.float32) + bo[...]


def _mlp_tc(hu, hi, W1, b1, W2, b2, W3, b3, Wo, bo):
    nb = hu.shape[0]
    full = lambda *shape: pl.BlockSpec(shape, lambda i: (0,) * len(shape))
    return pl.pallas_call(
        _mlp_body,
        grid=(nb // BT,),
        in_specs=[
            pl.BlockSpec((BT, D), lambda i: (i, 0)),
            pl.BlockSpec((BT, D), lambda i: (i, 0)),
            full(2 * D, H1),
            full(H1),
            full(H1, H2),
            full(H2),
            full(H2, H3),
            full(H3),
            full(H3, 1),
            full(1),
        ],
        out_specs=pl.BlockSpec((BT, 1), lambda i: (i, 0)),
        out_shape=jax.ShapeDtypeStruct((nb, 1), jnp.float32),
    )(hu, hi, W1, b1, W2, b2, W3, b3, Wo, bo)


def kernel(x_u, x_i, u_emb, i_emb, W1, b1, W2, b2, W3, b3, Wo, bo):
    x_u = x_u.astype(jnp.int32)
    x_i = x_i.astype(jnp.int32)
    outs = []
    off = 0
    for nb in CHUNKS:
        sl = slice(off, off + nb)
        hu, hi = _gather_sc(x_u[sl], x_i[sl], u_emb, i_emb, nb)
        outs.append(_mlp_tc(hu, hi, W1, b1, W2, b2, W3, b3, Wo, bo))
        off += nb
    return jnp.concatenate(outs, axis=0) if len(CHUNKS) > 1 else outs[0]


# speedup vs baseline: 1.0587x; 1.0086x over previous
import functools

import jax
import jax.numpy as jnp
from jax import lax
from jax.experimental import pallas as pl
from jax.experimental.pallas import tpu as pltpu
from jax.experimental.pallas import tpu_sc as plsc

B = 16384
D = 128
H1, H2, H3 = 1024, 512, 256

NC, NS = 2, 16
NW = NC * NS
CHUNK = 128

BT = 2048
CHUNKS = (8192, 8192)


def _gather_sc(x_u, x_i, u_emb, i_emb, nb):
    bpw = nb // NW
    nch = -(-bpw // CHUNK)
    assert bpw % nch == 0
    chunk = bpw // nch
    mesh = plsc.VectorSubcoreMesh(core_axis_name="c", subcore_axis_name="s")

    @functools.partial(
        pl.kernel,
        mesh=mesh,
        out_type=(
            jax.ShapeDtypeStruct((nb, D), jnp.float32),
            jax.ShapeDtypeStruct((nb, D), jnp.float32),
        ),
        scratch_types=[
            pltpu.VMEM((bpw,), jnp.int32),
            pltpu.VMEM((bpw, D), jnp.float32),
            pltpu.SemaphoreType.DMA,
        ],
        compiler_params=pltpu.CompilerParams(use_tc_tiling_on_sc=True),
    )
    def gk(xu_hbm, xi_hbm, ue_hbm, ie_hbm, hu_hbm, hi_hbm, idx_v, rows_v, sem):
        wid = lax.axis_index("s") * NC + lax.axis_index("c")
        base = wid * bpw
        for idx_hbm, tbl_hbm, out_hbm in (
            (xu_hbm, ue_hbm, hu_hbm),
            (xi_hbm, ie_hbm, hi_hbm),
        ):
            pltpu.sync_copy(idx_hbm.at[pl.ds(base, bpw)], idx_v)
            copies = []
            for j in range(nch):
                copies.append(pltpu.async_copy(
                    tbl_hbm.at[idx_v.at[pl.ds(j * chunk, chunk)]],
                    rows_v.at[pl.ds(j * chunk, chunk)],
                    sem,
                ))
            for c in copies:
                c.wait()
            pltpu.sync_copy(rows_v, out_hbm.at[pl.ds(base, bpw)])

    return gk(x_u, x_i, u_emb, i_emb)


def _mlp_body(hu, hi, w1, b1, w2, b2, w3, b3, wo, bo, out):
    h = jnp.concatenate([hu[...], hi[...]], axis=1)
    h1 = jnp.maximum(
        jnp.dot(h, w1[...], preferred_element_type=jnp.float32) + b1[...], 0.0)
    h2 = jnp.maximum(
        jnp.dot(h1, w2[...], preferred_element_type=jnp.float32) + b2[...], 0.0)
    h3 = jnp.maximum(
        jnp.dot(h2, w3[...], preferred_element_type=jnp.float32) + b3[...], 0.0)
    out[...] = jnp.dot(h3, wo[...], preferred_element_type=jnp.float32) + bo[...]


def _mlp_body_alias(hu, hi, w1, b1, w2, b2, w3, b3, wo, bo, prev, out):
    del prev
    _mlp_body(hu, hi, w1, b1, w2, b2, w3, b3, wo, bo, out)


def _mlp_tc(hu, hi, W1, b1, W2, b2, W3, b3, Wo, bo, off, prev):
    nb = hu.shape[0]
    off_b = off // BT
    full = lambda *shape: pl.BlockSpec(shape, lambda i: (0,) * len(shape))
    ins = [hu, hi, W1, b1, W2, b2, W3, b3, Wo, bo]
    in_specs = [
        pl.BlockSpec((BT, D), lambda i: (i, 0)),
        pl.BlockSpec((BT, D), lambda i: (i, 0)),
        full(2 * D, H1),
        full(H1),
        full(H1, H2),
        full(H2),
        full(H2, H3),
        full(H3),
        full(H3, 1),
        full(1),
    ]
    kwargs = {}
    body = _mlp_body
    if prev is not None:
        ins.append(prev)
        in_specs.append(pl.BlockSpec(memory_space=pl.ANY))
        kwargs["input_output_aliases"] = {10: 0}
        body = _mlp_body_alias
    return pl.pallas_call(
        body,
        grid=(nb // BT,),
        in_specs=in_specs,
        out_specs=pl.BlockSpec((BT, 1), lambda i: (i + off_b, 0)),
        out_shape=jax.ShapeDtypeStruct((B, 1), jnp.float32),
        **kwargs,
    )(*ins)


def kernel(x_u, x_i, u_emb, i_emb, W1, b1, W2, b2, W3, b3, Wo, bo):
    x_u = x_u.astype(jnp.int32)
    x_i = x_i.astype(jnp.int32)
    out = None
    off = 0
    for nb in CHUNKS:
        sl = slice(off, off + nb)
        hu, hi = _gather_sc(x_u[sl], x_i[sl], u_emb, i_emb, nb)
        out = _mlp_tc(hu, hi, W1, b1, W2, b2, W3, b3, Wo, bo, off, out)
        off += nb
    return out
